# baseline (speedup 1.0000x reference)
import jax
import jax.numpy as jnp
from jax import lax
from jax.experimental import pallas as pl
from jax.experimental.pallas import tpu as pltpu
from jax.experimental.pallas import tpu_sc as plsc

N = 10000
E = 320000
D = 128
NPAD = 10240

NC = 2
NS = 16
NW = NC * NS
EPT = 9984
REM = E - NW * EPT
XW = REM // 128
BPT = NPAD // NS


def _sc_hist_body(ei_hbm, ones_hbm, out_hbm, iv2, ivx, iv_s, iv_d, ivx_s,
                  ivx_d, ones_v, zero_v, obuf_v, sem, sem2, hist_src,
                  hist_dst):
    c = lax.axis_index("c")
    s = lax.axis_index("s")
    w = c * NS + s

    load_m = pltpu.async_copy(ei_hbm.at[:, pl.ds(w * EPT, EPT)], iv2, sem)
    load_x = pltpu.async_copy(
        ei_hbm.at[:, pl.ds(NW * EPT + lax.min(w, XW - 1) * 128, 128)],
        ivx, sem)
    load_o = pltpu.async_copy(ones_hbm, ones_v, sem)

    @pl.loop(0, BPT // 16)
    def _(i):
        zero_v[pl.ds(i * 16, 16)] = jnp.zeros((16,), jnp.float32)

    pltpu.sync_copy(zero_v, hist_src.at[pl.ds(s * BPT, BPT)])
    pltpu.sync_copy(zero_v, hist_dst.at[pl.ds(s * BPT, BPT)])

    load_m.wait()

    @pl.loop(0, EPT // 16, unroll=8)
    def _(i):
        sl = pl.ds(i * 16, 16)
        iv_s[sl] = iv2[0, sl]

    plsc.subcore_barrier()

    load_o.wait()
    sc_s = pltpu.async_copy(ones_v, hist_src.at[iv_s], sem2, add=True)

    @pl.loop(0, EPT // 16, unroll=8)
    def _(i):
        sl = pl.ds(i * 16, 16)
        iv_d[sl] = iv2[1, sl]

    sc_d = pltpu.async_copy(ones_v, hist_dst.at[iv_d], sem2, add=True)

    load_x.wait()

    @pl.when(w < XW)
    def _():
        @pl.loop(0, 8)
        def _(i):
            sl = pl.ds(i * 16, 16)
            ivx_s[sl] = ivx[0, sl]
            ivx_d[sl] = ivx[1, sl]

        pltpu.sync_copy(ones_v.at[pl.ds(0, 128)],
                        hist_src.at[ivx_s], add=True)
        pltpu.sync_copy(ones_v.at[pl.ds(0, 128)],
                        hist_dst.at[ivx_d], add=True)

    sc_s.wait()
    sc_d.wait()
    plsc.subcore_barrier()

    pltpu.sync_copy(hist_src.at[pl.ds(s * BPT, BPT)], obuf_v)
    pltpu.sync_copy(obuf_v, out_hbm.at[c, 0, pl.ds(s * BPT, BPT)])
    pltpu.sync_copy(hist_dst.at[pl.ds(s * BPT, BPT)], obuf_v)
    pltpu.sync_copy(obuf_v, out_hbm.at[c, 1, pl.ds(s * BPT, BPT)])


_sc_hist = pl.kernel(
    _sc_hist_body,
    out_type=jax.ShapeDtypeStruct((NC, 2, NPAD), jnp.float32),
    mesh=plsc.VectorSubcoreMesh(core_axis_name="c", subcore_axis_name="s",
                                num_cores=NC, num_subcores=NS),
    scratch_types=[
        pltpu.VMEM((2, EPT), jnp.int32),
        pltpu.VMEM((2, 128), jnp.int32),
        pltpu.VMEM((EPT,), jnp.int32),
        pltpu.VMEM((EPT,), jnp.int32),
        pltpu.VMEM((128,), jnp.int32),
        pltpu.VMEM((128,), jnp.int32),
        pltpu.VMEM((EPT,), jnp.float32),
        pltpu.VMEM((BPT,), jnp.float32),
        pltpu.VMEM((BPT,), jnp.float32),
        pltpu.SemaphoreType.DMA,
        pltpu.SemaphoreType.DMA,
        pltpu.VMEM_SHARED((NPAD,), jnp.float32),
        pltpu.VMEM_SHARED((NPAD,), jnp.float32),
    ],
)


RB = 2000


def _tc_scale_body(x_ref, p_ref, o_ref):
    p = p_ref[...].reshape(NC, 2, RB)
    out_deg = p[0, 0] + p[1, 0]
    in_deg = p[0, 1] + p[1, 1]
    div = jnp.where(in_deg > 0.0, jnp.maximum(out_deg, 1.0), 1.0)
    o_ref[...] = x_ref[...] * (1.0 / div)[:, None]


@jax.jit
def kernel(x, edge_index, edge_type, w, b, edge_embeddings):
    del edge_type, w, b, edge_embeddings
    ei = edge_index.astype(jnp.int32)
    ones = jnp.ones((EPT,), jnp.float32)

    partial = _sc_hist(ei, ones)

    p5 = partial[:, :, :N].reshape(NC, 2, N // RB, 8, RB // 8)

    return pl.pallas_call(
        _tc_scale_body,
        grid=(N // RB,),
        in_specs=[
            pl.BlockSpec((RB, D), lambda i: (i, 0)),
            pl.BlockSpec((NC, 2, 1, 8, RB // 8), lambda i: (0, 0, i, 0, 0)),
        ],
        out_specs=pl.BlockSpec((RB, D), lambda i: (i, 0)),
        out_shape=jax.ShapeDtypeStruct((N, D), jnp.float32),
    )(x, p5)


# speedup vs baseline: 1.0813x; 1.0813x over previous
import jax
import jax.numpy as jnp
from jax import lax
from jax.experimental import pallas as pl
from jax.experimental.pallas import tpu as pltpu
from jax.experimental.pallas import tpu_sc as plsc

N = 10000
E = 320000
D = 128
NPAD = 10240

NC = 2
NS = 16
NW = NC * NS
EPT = 9984
REM = E - NW * EPT
XW = REM // 128
BPT = NPAD // NS


def _sc_hist_body(ei_hbm, ones_hbm, out_hbm, iv2, ivx, iv_s, iv_d, ivx_s,
                  ivx_d, ones_v, zero_v, obuf_v, sem, sem2, hist_src,
                  hist_dst):
    c = lax.axis_index("c")
    s = lax.axis_index("s")
    w = c * NS + s

    load_m = pltpu.async_copy(ei_hbm.at[:, pl.ds(w * EPT, EPT)], iv2, sem)
    load_x = pltpu.async_copy(
        ei_hbm.at[:, pl.ds(NW * EPT + lax.min(w, XW - 1) * 128, 128)],
        ivx, sem)
    load_o = pltpu.async_copy(ones_hbm, ones_v, sem)

    @pl.loop(0, BPT // 16)
    def _(i):
        zero_v[pl.ds(i * 16, 16)] = jnp.zeros((16,), jnp.float32)

    pltpu.sync_copy(zero_v, hist_src.at[pl.ds(s * BPT, BPT)])
    pltpu.sync_copy(zero_v, hist_dst.at[pl.ds(s * BPT, BPT)])

    load_m.wait()

    @pl.loop(0, EPT // 16, unroll=8)
    def _(i):
        sl = pl.ds(i * 16, 16)
        iv_s[sl] = iv2[0, sl]

    plsc.subcore_barrier()

    load_o.wait()
    sc_s = pltpu.async_copy(ones_v, hist_src.at[iv_s], sem2, add=True)

    @pl.loop(0, EPT // 16, unroll=8)
    def _(i):
        sl = pl.ds(i * 16, 16)
        iv_d[sl] = iv2[1, sl]

    sc_d = pltpu.async_copy(ones_v, hist_dst.at[iv_d], sem2, add=True)

    load_x.wait()

    @pl.when(w < XW)
    def _():
        @pl.loop(0, 8)
        def _(i):
            sl = pl.ds(i * 16, 16)
            ivx_s[sl] = ivx[0, sl]
            ivx_d[sl] = ivx[1, sl]

        pltpu.sync_copy(ones_v.at[pl.ds(0, 128)],
                        hist_src.at[ivx_s], add=True)
        pltpu.sync_copy(ones_v.at[pl.ds(0, 128)],
                        hist_dst.at[ivx_d], add=True)

    sc_s.wait()
    sc_d.wait()
    plsc.subcore_barrier()

    pltpu.sync_copy(hist_src.at[pl.ds(s * BPT, BPT)], obuf_v)
    pltpu.sync_copy(obuf_v, out_hbm.at[c, 0, pl.ds(s * BPT, BPT)])
    pltpu.sync_copy(hist_dst.at[pl.ds(s * BPT, BPT)], obuf_v)
    pltpu.sync_copy(obuf_v, out_hbm.at[c, 1, pl.ds(s * BPT, BPT)])


_sc_hist = pl.kernel(
    _sc_hist_body,
    out_type=jax.ShapeDtypeStruct((NC, 2, NPAD), jnp.float32),
    mesh=plsc.VectorSubcoreMesh(core_axis_name="c", subcore_axis_name="s",
                                num_cores=NC, num_subcores=NS),
    scratch_types=[
        pltpu.VMEM((2, EPT), jnp.int32),
        pltpu.VMEM((2, 128), jnp.int32),
        pltpu.VMEM((EPT,), jnp.int32),
        pltpu.VMEM((EPT,), jnp.int32),
        pltpu.VMEM((128,), jnp.int32),
        pltpu.VMEM((128,), jnp.int32),
        pltpu.VMEM((EPT,), jnp.float32),
        pltpu.VMEM((BPT,), jnp.float32),
        pltpu.VMEM((BPT,), jnp.float32),
        pltpu.SemaphoreType.DMA,
        pltpu.SemaphoreType.DMA,
        pltpu.VMEM_SHARED((NPAD,), jnp.float32),
        pltpu.VMEM_SHARED((NPAD,), jnp.float32),
    ],
)


def _tc_scale_body(x_ref, p_ref, o_ref):
    p = p_ref[...]
    out_deg = p[0, 0] + p[1, 0]
    in_deg = p[0, 1] + p[1, 1]
    div = jnp.where(in_deg > 0.0, jnp.maximum(out_deg, 1.0), 1.0)
    recip = (1.0 / div)[:N]
    o_ref[...] = x_ref[...] * recip[:, None]


@jax.jit
def kernel(x, edge_index, edge_type, w, b, edge_embeddings):
    del edge_type, w, b, edge_embeddings
    ei = edge_index.astype(jnp.int32)
    ones = jnp.ones((EPT,), jnp.float32)

    partial = _sc_hist(ei, ones)

    return pl.pallas_call(
        _tc_scale_body,
        out_shape=jax.ShapeDtypeStruct((N, D), jnp.float32),
    )(x, partial)
